# baseline (speedup 1.0000x reference)
import jax
import jax.numpy as jnp
import numpy as np
from jax.experimental import pallas as pl
from jax.experimental.pallas import tpu as pltpu

N = 4096
M = 1639
MP = 1664
K = 20
B = 8
R = 512
NB = N // R


def _threefry2x32(k0, k1, x0, x1):
    def rotl(x, r):
        return ((x << np.uint32(r)) | (x >> np.uint32(32 - r))).astype(np.uint32)

    ks2 = np.uint32(0x1BD11BDA) ^ k0 ^ k1
    rotations = ([13, 15, 26, 6], [17, 29, 16, 24])
    x0 = (x0 + k0).astype(np.uint32)
    x1 = (x1 + k1).astype(np.uint32)
    adds = [(k1, ks2), (ks2, k0), (k0, k1), (k1, ks2), (ks2, k0)]
    for g in range(5):
        for r in rotations[g % 2]:
            x0 = (x0 + x1).astype(np.uint32)
            x1 = rotl(x1, r)
            x1 = x1 ^ x0
        a0, a1 = adds[g]
        x0 = (x0 + a0).astype(np.uint32)
        x1 = (x1 + a1 + np.uint32(g + 1)).astype(np.uint32)
    return x0, x1


def _noise_padded() -> np.ndarray:
    n = N * M
    b1, b2 = _threefry2x32(
        np.uint32(0), np.uint32(42), np.zeros(n, np.uint32), np.arange(n, dtype=np.uint32)
    )
    bits = b1 ^ b2
    z = ((bits >> np.uint32(9)) | np.uint32(0x3F800000)).view(np.float32)
    z = (z - np.float32(1.0)) * np.float32(0.01)
    out = np.full((N, MP), -1.0, np.float32)
    out[:, :M] = z.reshape(N, M)
    return out


_NOISE = _noise_padded()


def _topk_kernel(nv_ref, ev_ref, noise_ref, he_ref, hew_ref):
    nv = nv_ref[...]
    ev = ev_ref[...]
    de = jnp.tanh(2.0 * nv)
    ee = jnp.tanh(2.0 * ev)
    adj = jax.lax.dot_general(
        de, ee, (((1,), (1,)), ((), ())), preferred_element_type=jnp.float32
    )
    adj = jax.nn.relu(jnp.tanh(2.0 * adj))
    noise = noise_ref[...]
    a = adj + noise
    iota = jax.lax.broadcasted_iota(jnp.int32, (R, MP), 1)
    q8 = jax.lax.convert_element_type(noise * 25600.0, jnp.int32) & 255
    pk = (iota << 8) | q8

    H = 768
    aL, aR, aS = a[:, :H], a[:, H : 2 * H], a[:, 2 * H :]
    pkL, pkR, pkS = pk[:, :H], pk[:, H : 2 * H], pk[:, 2 * H :]
    swap = aR > aL
    bv = jnp.concatenate([jnp.where(swap, aR, aL), aS], axis=1)
    bpk = jnp.concatenate([jnp.where(swap, pkR, pkL), pkS], axis=1)
    cv = jnp.concatenate(
        [jnp.where(swap, aL, aR), jnp.full((R, MP - 2 * H), -2.0, jnp.float32)], axis=1
    )
    cpk = jnp.concatenate(
        [jnp.where(swap, pkL, pkR), jnp.full((R, MP - 2 * H), -1, jnp.int32)], axis=1
    )

    kio = jax.lax.broadcasted_iota(jnp.int32, (R, K), 1)
    idxpk = jnp.zeros((R, K), jnp.int32)
    mm = jnp.zeros((R, K), jnp.float32)
    for k in range(K):
        m = jnp.max(bv, axis=1, keepdims=True)
        jpk = jnp.min(jnp.where(bv == m, bpk, 1 << 30), axis=1, keepdims=True)
        sel = bpk == jpk
        bv = jnp.where(sel, cv, bv)
        bpk = jnp.where(sel, cpk, bpk)
        cv = jnp.where(sel, -2.0, cv)
        selk = kio == k
        idxpk = jnp.where(selk, jpk, idxpk)
        mm = jnp.where(selk, m, mm)

    idx = idxpk >> 8
    val = mm - (idxpk & 255).astype(jnp.float32) * (0.01 / 256.0)

    rank = jnp.zeros((R, K), jnp.int32)
    for l in range(K):
        rank = rank + (idx[:, l : l + 1] < idx).astype(jnp.int32)
    sidx = jnp.zeros((R, K), jnp.int32)
    sval = jnp.zeros((R, K), jnp.float32)
    for p in range(K):
        sel = kio == rank[:, p : p + 1]
        sidx = jnp.where(sel, idx[:, p : p + 1], sidx)
        sval = jnp.where(sel, val[:, p : p + 1], sval)

    row0 = pl.program_id(0) * R
    rows = row0 + jax.lax.broadcasted_iota(jnp.int32, (R, K), 0)
    for i in range(B):
        he_ref[0, i] = rows + i * N
        he_ref[1, i] = sidx + i * M
        hew_ref[i] = sval


_HE0 = np.tile(np.repeat(np.arange(N, dtype=np.int32), K), B) + np.repeat(
    np.arange(B, dtype=np.int32) * N, N * K
)


def kernel(x, nodevec, edgevec):
    del x
    ev = jnp.zeros((MP, 40), jnp.float32).at[:M].set(edgevec)
    noise = jnp.asarray(_NOISE)
    he, hew = pl.pallas_call(
        _topk_kernel,
        grid=(NB,),
        in_specs=[
            pl.BlockSpec((R, 40), lambda b: (b, 0)),
            pl.BlockSpec((MP, 40), lambda b: (0, 0)),
            pl.BlockSpec((R, MP), lambda b: (b, 0)),
        ],
        out_specs=[
            pl.BlockSpec((2, B, R, K), lambda b: (0, 0, b, 0)),
            pl.BlockSpec((B, R, K), lambda b: (0, b, 0)),
        ],
        out_shape=[
            jax.ShapeDtypeStruct((2, B, N, K), jnp.int32),
            jax.ShapeDtypeStruct((B, N, K), jnp.float32),
        ],
        compiler_params=pltpu.CompilerParams(
            dimension_semantics=("parallel",),
        ),
    )(nodevec, ev, noise)
    return (he.reshape(2, B * N * K), hew.reshape(B * N * K))


# speedup vs baseline: 1.2776x; 1.2776x over previous
import jax
import jax.numpy as jnp
import numpy as np
from jax.experimental import pallas as pl
from jax.experimental.pallas import tpu as pltpu

N = 4096
M = 1639
MP = 1664
K = 20
B = 8
R = 512
NB = N // R


def _threefry2x32(k0, k1, x0, x1):
    def rotl(x, r):
        return ((x << np.uint32(r)) | (x >> np.uint32(32 - r))).astype(np.uint32)

    ks2 = np.uint32(0x1BD11BDA) ^ k0 ^ k1
    rotations = ([13, 15, 26, 6], [17, 29, 16, 24])
    x0 = (x0 + k0).astype(np.uint32)
    x1 = (x1 + k1).astype(np.uint32)
    adds = [(k1, ks2), (ks2, k0), (k0, k1), (k1, ks2), (ks2, k0)]
    for g in range(5):
        for r in rotations[g % 2]:
            x0 = (x0 + x1).astype(np.uint32)
            x1 = rotl(x1, r)
            x1 = x1 ^ x0
        a0, a1 = adds[g]
        x0 = (x0 + a0).astype(np.uint32)
        x1 = (x1 + a1 + np.uint32(g + 1)).astype(np.uint32)
    return x0, x1


def _noise_padded() -> np.ndarray:
    n = N * M
    b1, b2 = _threefry2x32(
        np.uint32(0), np.uint32(42), np.zeros(n, np.uint32), np.arange(n, dtype=np.uint32)
    )
    bits = b1 ^ b2
    z = ((bits >> np.uint32(9)) | np.uint32(0x3F800000)).view(np.float32)
    z = (z - np.float32(1.0)) * np.float32(0.01)
    out = np.full((N, MP), -1.0, np.float32)
    out[:, :M] = z.reshape(N, M)
    return out


_NOISE = _noise_padded()
_PK = (np.arange(MP, dtype=np.int32)[None, :] << 8) | (
    (_NOISE * 25600.0).astype(np.int64).astype(np.int32) & 255
)
_PK = np.ascontiguousarray(np.broadcast_to(_PK, (N, MP)) | 0)


def _topk_kernel(nv_ref, ev_ref, noise_ref, pk_ref, sidx_ref, sval_ref):
    nv = nv_ref[...]
    ev = ev_ref[...]
    de = jnp.tanh(2.0 * nv)
    ee = jnp.tanh(2.0 * ev)
    adj = jax.lax.dot_general(
        de, ee, (((1,), (1,)), ((), ())), preferred_element_type=jnp.float32
    )
    adj = jax.nn.relu(jnp.tanh(2.0 * adj))
    a = adj + noise_ref[...]
    pk = pk_ref[...]

    H = 768
    aL, aR, aS = a[:, :H], a[:, H : 2 * H], a[:, 2 * H :]
    pkL, pkR, pkS = pk[:, :H], pk[:, H : 2 * H], pk[:, 2 * H :]
    swap = aR > aL
    bv = jnp.concatenate([jnp.where(swap, aR, aL), aS], axis=1)
    bpk = jnp.concatenate([jnp.where(swap, pkR, pkL), pkS], axis=1)
    cv = jnp.concatenate(
        [jnp.where(swap, aL, aR), jnp.full((R, MP - 2 * H), -2.0, jnp.float32)], axis=1
    )
    cpk = jnp.concatenate(
        [jnp.where(swap, pkL, pkR), jnp.full((R, MP - 2 * H), -1, jnp.int32)], axis=1
    )

    kio = jax.lax.broadcasted_iota(jnp.int32, (R, K), 1)
    idxpk = jnp.zeros((R, K), jnp.int32)
    mm = jnp.zeros((R, K), jnp.float32)
    for k in range(K):
        m = jnp.max(bv, axis=1, keepdims=True)
        jpk = jnp.min(jnp.where(bv == m, bpk, 1 << 30), axis=1, keepdims=True)
        sel = bpk == jpk
        bv = jnp.where(sel, cv, bv)
        bpk = jnp.where(sel, cpk, bpk)
        cv = jnp.where(sel, -2.0, cv)
        selk = kio == k
        idxpk = jnp.where(selk, jpk, idxpk)
        mm = jnp.where(selk, m, mm)

    idx = idxpk >> 8
    val = mm - (idxpk & 255).astype(jnp.float32) * (0.01 / 256.0)

    rank = jnp.zeros((R, K), jnp.int32)
    for l in range(K):
        rank = rank + (idx[:, l : l + 1] < idx).astype(jnp.int32)
    sidx = jnp.zeros((R, K), jnp.int32)
    sval = jnp.zeros((R, K), jnp.float32)
    for p in range(K):
        sel = kio == rank[:, p : p + 1]
        sidx = jnp.where(sel, idx[:, p : p + 1], sidx)
        sval = jnp.where(sel, val[:, p : p + 1], sval)

    sidx_ref[...] = sidx
    sval_ref[...] = sval


_HE0 = np.tile(np.repeat(np.arange(N, dtype=np.int32), K), B) + np.repeat(
    np.arange(B, dtype=np.int32) * N, N * K
)


def kernel(x, nodevec, edgevec):
    del x
    ev = jnp.zeros((MP, 40), jnp.float32).at[:M].set(edgevec)
    noise = jnp.asarray(_NOISE)
    sidx, sval = pl.pallas_call(
        _topk_kernel,
        grid=(NB,),
        in_specs=[
            pl.BlockSpec((R, 40), lambda b: (b, 0)),
            pl.BlockSpec((MP, 40), lambda b: (0, 0)),
            pl.BlockSpec((R, MP), lambda b: (b, 0)),
            pl.BlockSpec((R, MP), lambda b: (b, 0)),
        ],
        out_specs=[
            pl.BlockSpec((R, K), lambda b: (b, 0)),
            pl.BlockSpec((R, K), lambda b: (b, 0)),
        ],
        out_shape=[
            jax.ShapeDtypeStruct((N, K), jnp.int32),
            jax.ShapeDtypeStruct((N, K), jnp.float32),
        ],
        compiler_params=pltpu.CompilerParams(
            dimension_semantics=("parallel",),
        ),
    )(nodevec, ev, noise, jnp.asarray(_PK))
    he1 = (sidx.reshape(1, N * K) + (jnp.arange(B, dtype=jnp.int32) * M)[:, None]).reshape(-1)
    HE = jnp.stack([jnp.asarray(_HE0), he1], axis=0)
    HEW = jnp.broadcast_to(sval.reshape(1, N * K), (B, N * K)).reshape(-1)
    return (HE, HEW)


# speedup vs baseline: 1.5982x; 1.2509x over previous
import jax
import jax.numpy as jnp
import numpy as np
from jax.experimental import pallas as pl
from jax.experimental.pallas import tpu as pltpu

N = 4096
M = 1639
MP = 1664
K = 20
B = 8
R = 512
NB = N // R


def _threefry2x32(k0, k1, x0, x1):
    def rotl(x, r):
        return ((x << np.uint32(r)) | (x >> np.uint32(32 - r))).astype(np.uint32)

    ks2 = np.uint32(0x1BD11BDA) ^ k0 ^ k1
    rotations = ([13, 15, 26, 6], [17, 29, 16, 24])
    x0 = (x0 + k0).astype(np.uint32)
    x1 = (x1 + k1).astype(np.uint32)
    adds = [(k1, ks2), (ks2, k0), (k0, k1), (k1, ks2), (ks2, k0)]
    for g in range(5):
        for r in rotations[g % 2]:
            x0 = (x0 + x1).astype(np.uint32)
            x1 = rotl(x1, r)
            x1 = x1 ^ x0
        a0, a1 = adds[g]
        x0 = (x0 + a0).astype(np.uint32)
        x1 = (x1 + a1 + np.uint32(g + 1)).astype(np.uint32)
    return x0, x1


def _noise_padded() -> np.ndarray:
    n = N * M
    b1, b2 = _threefry2x32(
        np.uint32(0), np.uint32(42), np.zeros(n, np.uint32), np.arange(n, dtype=np.uint32)
    )
    bits = b1 ^ b2
    z = ((bits >> np.uint32(9)) | np.uint32(0x3F800000)).view(np.float32)
    z = (z - np.float32(1.0)) * np.float32(0.01)
    out = np.full((N, MP), -1.0, np.float32)
    out[:, :M] = z.reshape(N, M)
    return out


_NOISE = _noise_padded()


def _topk_kernel(nv_ref, ev_ref, noise_ref, sidx_ref, sval_ref):
    nv = nv_ref[...]
    ev = ev_ref[...]
    de = jnp.tanh(2.0 * nv)
    ee = jnp.tanh(2.0 * ev)
    adj = jax.lax.dot_general(
        de, ee, (((1,), (1,)), ((), ())), preferred_element_type=jnp.float32
    )
    adj = jax.nn.relu(jnp.tanh(2.0 * adj))
    noise = noise_ref[...]
    a = adj + noise
    iota = jax.lax.broadcasted_iota(jnp.int32, (R, MP), 1)
    q8 = jax.lax.convert_element_type(noise * 25600.0, jnp.int32) & 255
    pk = (iota << 8) | q8

    H = 768
    aL, aR, aS = a[:, :H], a[:, H : 2 * H], a[:, 2 * H :]
    pkL, pkR, pkS = pk[:, :H], pk[:, H : 2 * H], pk[:, 2 * H :]
    swap = aR > aL
    bv = jnp.concatenate([jnp.where(swap, aR, aL), aS], axis=1)
    bpk = jnp.concatenate([jnp.where(swap, pkR, pkL), pkS], axis=1)
    cv = jnp.concatenate(
        [jnp.where(swap, aL, aR), jnp.full((R, MP - 2 * H), -2.0, jnp.float32)], axis=1
    )
    cpk = jnp.concatenate(
        [jnp.where(swap, pkL, pkR), jnp.full((R, MP - 2 * H), -1, jnp.int32)], axis=1
    )

    kio = jax.lax.broadcasted_iota(jnp.int32, (R, K), 1)
    idxpk = jnp.zeros((R, K), jnp.int32)
    mm = jnp.zeros((R, K), jnp.float32)
    for k in range(K):
        m = jnp.max(bv, axis=1, keepdims=True)
        jpk = jnp.min(jnp.where(bv == m, bpk, 1 << 30), axis=1, keepdims=True)
        sel = bpk == jpk
        bv = jnp.where(sel, cv, bv)
        bpk = jnp.where(sel, cpk, bpk)
        cv = jnp.where(sel, -2.0, cv)
        selk = kio == k
        idxpk = jnp.where(selk, jpk, idxpk)
        mm = jnp.where(selk, m, mm)

    idx = idxpk >> 8
    val = mm - (idxpk & 255).astype(jnp.float32) * (0.01 / 256.0)

    idxT = idx.T
    valT = val.T
    rankT = jnp.zeros((K, R), jnp.int32)
    for l in range(K):
        rankT = rankT + (idxT[l : l + 1, :] < idxT).astype(jnp.int32)
    kioT = jax.lax.broadcasted_iota(jnp.int32, (K, R), 0)
    sidxT = jnp.zeros((K, R), jnp.int32)
    svalT = jnp.zeros((K, R), jnp.float32)
    for p in range(K):
        sel = kioT == rankT[p : p + 1, :]
        sidxT = jnp.where(sel, idxT[p : p + 1, :], sidxT)
        svalT = jnp.where(sel, valT[p : p + 1, :], svalT)
    sidx = sidxT.T
    sval = svalT.T

    sidx_ref[...] = sidx
    sval_ref[...] = sval


_HE0 = np.tile(np.repeat(np.arange(N, dtype=np.int32), K), B) + np.repeat(
    np.arange(B, dtype=np.int32) * N, N * K
)


def kernel(x, nodevec, edgevec):
    del x
    ev = jnp.zeros((MP, 40), jnp.float32).at[:M].set(edgevec)
    noise = jnp.asarray(_NOISE)
    sidx, sval = pl.pallas_call(
        _topk_kernel,
        grid=(NB,),
        in_specs=[
            pl.BlockSpec((R, 40), lambda b: (b, 0)),
            pl.BlockSpec((MP, 40), lambda b: (0, 0)),
            pl.BlockSpec((R, MP), lambda b: (b, 0)),
        ],
        out_specs=[
            pl.BlockSpec((R, K), lambda b: (b, 0)),
            pl.BlockSpec((R, K), lambda b: (b, 0)),
        ],
        out_shape=[
            jax.ShapeDtypeStruct((N, K), jnp.int32),
            jax.ShapeDtypeStruct((N, K), jnp.float32),
        ],
        compiler_params=pltpu.CompilerParams(
            dimension_semantics=("parallel",),
        ),
    )(nodevec, ev, noise)
    he1 = (sidx.reshape(1, N * K) + (jnp.arange(B, dtype=jnp.int32) * M)[:, None]).reshape(-1)
    HE = jnp.stack([jnp.asarray(_HE0), he1], axis=0)
    HEW = jnp.broadcast_to(sval.reshape(1, N * K), (B, N * K)).reshape(-1)
    return (HE, HEW)
